# baseline (speedup 1.0000x reference)
import functools

import jax
import jax.numpy as jnp
from jax import lax
from jax.experimental import pallas as pl
from jax.experimental.pallas import tpu as pltpu
from jax.experimental.pallas import tpu_sc as plsc

NC = 2
NS = 16
NW = NC * NS
CHUNK = 2000


def _unpack_lo(w):
    return plsc.bitcast(w << 16, jnp.float32)


def _unpack_hi(w):
    return plsc.bitcast((w >> 16) << 16, jnp.float32)


def _bond_kernel(nchunks, xy_hbm, z_hbm, pairs_hbm, r0_hbm, k_hbm,
                 out_hbm, stage_hbm,
                 table_v,
                 pr0_v, dd0_v, r00_v, k0_v,
                 pr1_v, dd1_v, r01_v, k1_v,
                 acc_v, sem0, sem1, osem0, osem1):
    cid = lax.axis_index("c")
    sid = lax.axis_index("s")
    wid = sid * NC + cid
    base_e = wid * nchunks * CHUNK

    lane2 = lax.iota(jnp.int32, 16) * 2

    bufs = (
        (pr0_v, dd0_v, r00_v, k0_v, sem0, osem0),
        (pr1_v, dd1_v, r01_v, k1_v, sem1, osem1),
    )

    pltpu.sync_copy(xy_hbm, table_v)

    def issue1(ch, b):
        pr_v, _, _, _, sem, _ = bufs[b]
        eb = base_e + ch * CHUNK
        pltpu.async_copy(pairs_hbm.at[pl.ds(2 * eb, 2 * CHUNK)], pr_v, sem)

    def drain1(b):
        pr_v, _, _, _, sem, _ = bufs[b]
        pltpu.make_async_copy(pairs_hbm.at[pl.ds(0, 2 * CHUNK)], pr_v,
                              sem).wait()

    def compute1(ch, b):
        pr_v, dd_v, _, _, _, osem = bufs[b]
        eb = base_e + ch * CHUNK

        @plsc.parallel_loop(0, CHUNK // 16, unroll=4)
        def vloop(v):
            e0 = pl.multiple_of(v * 16, 16)
            p0 = e0 * 2 + lane2
            iv = plsc.load_gather(pr_v, [p0])
            jv = plsc.load_gather(pr_v, [p0 + 1])
            wi = plsc.load_gather(table_v, [iv])
            wj = plsc.load_gather(table_v, [jv])
            dx = _unpack_lo(wi) - _unpack_lo(wj)
            dy = _unpack_hi(wi) - _unpack_hi(wj)
            dd_v[pl.ds(e0, 16)] = dx * dx + dy * dy

        pltpu.async_copy(dd_v, stage_hbm.at[pl.ds(eb, CHUNK)], osem)

    def drain_out(b):
        _, dd_v, _, _, _, osem = bufs[b]
        pltpu.make_async_copy(stage_hbm.at[pl.ds(0, CHUNK)], dd_v,
                              osem).wait()

    issue1(0, 0)
    issue1(1, 1)

    def body1(p, carry):
        drain1(0)
        compute1(2 * p, 0)
        issue1(2 * p + 2, 0)
        drain1(1)
        compute1(2 * p + 1, 1)
        issue1(2 * p + 3, 1)
        drain_out(0)
        drain_out(1)
        return carry

    lax.fori_loop(0, nchunks // 2 - 1, body1, jnp.int32(0))
    drain1(0)
    compute1(nchunks - 2, 0)
    drain1(1)
    compute1(nchunks - 1, 1)
    drain_out(0)
    drain_out(1)

    pltpu.sync_copy(z_hbm, table_v)

    def issue2(ch, b):
        pr_v, dd_v, r0_v, k_v, sem, _ = bufs[b]
        eb = base_e + ch * CHUNK
        pltpu.async_copy(pairs_hbm.at[pl.ds(2 * eb, 2 * CHUNK)], pr_v, sem)
        pltpu.async_copy(stage_hbm.at[pl.ds(eb, CHUNK)], dd_v, sem)
        pltpu.async_copy(r0_hbm.at[pl.ds(eb, CHUNK)], r0_v, sem)
        pltpu.async_copy(k_hbm.at[pl.ds(eb, CHUNK)], k_v, sem)

    def drain2(b):
        pr_v, dd_v, r0_v, k_v, sem, _ = bufs[b]
        pltpu.make_async_copy(pairs_hbm.at[pl.ds(0, 2 * CHUNK)], pr_v,
                              sem).wait()
        pltpu.make_async_copy(stage_hbm.at[pl.ds(0, CHUNK)], dd_v,
                              sem).wait()
        pltpu.make_async_copy(r0_hbm.at[pl.ds(0, CHUNK)], r0_v, sem).wait()
        pltpu.make_async_copy(k_hbm.at[pl.ds(0, CHUNK)], k_v, sem).wait()

    def compute2(b, acc):
        pr_v, dd_v, r0_v, k_v, _, _ = bufs[b]

        @plsc.parallel_loop(0, CHUNK // 16, unroll=4, carry=acc)
        def vloop(v, acc):
            e0 = pl.multiple_of(v * 16, 16)
            p0 = e0 * 2 + lane2
            iv = plsc.load_gather(pr_v, [p0])
            jv = plsc.load_gather(pr_v, [p0 + 1])
            zi = plsc.bitcast(plsc.load_gather(table_v, [iv]), jnp.float32)
            zj = plsc.bitcast(plsc.load_gather(table_v, [jv]), jnp.float32)
            dz = zi - zj
            d2 = dd_v[pl.ds(e0, 16)] + dz * dz
            d2 = jnp.maximum(d2, 1e-12)
            ib = plsc.bitcast(d2, jnp.int32)
            y = plsc.bitcast(jnp.int32(0x5F3759DF) - (ib >> 1), jnp.float32)
            hx = 0.5 * d2
            y = y * (1.5 - hx * y * y)
            y = y * (1.5 - hx * y * y)
            r = d2 * y
            t = r - r0_v[pl.ds(e0, 16)]
            return acc + (t * t) * k_v[pl.ds(e0, 16)]

        return vloop

    issue2(0, 0)
    issue2(1, 1)

    def body2(p, acc):
        drain2(0)
        acc = compute2(0, acc)
        issue2(2 * p + 2, 0)
        drain2(1)
        acc = compute2(1, acc)
        issue2(2 * p + 3, 1)
        return acc

    acc = lax.fori_loop(0, nchunks // 2 - 1, body2,
                        jnp.zeros((16,), jnp.float32))
    drain2(0)
    acc = compute2(0, acc)
    drain2(1)
    acc = compute2(1, acc)

    acc_v[...] = acc * 0.5
    pltpu.sync_copy(acc_v, out_hbm.at[wid])


def kernel(coords, pairs, r0, k):
    e = pairs.shape[0]
    n = coords.shape[0]
    r0 = r0.astype(jnp.float32)
    k = k.astype(jnp.float32)
    pairs_flat = pairs.astype(jnp.int32).reshape(2 * e)

    grain = NW * CHUNK * 2
    e_pad = ((e + grain - 1) // grain) * grain
    pad = e_pad - e
    if pad:
        pairs_flat = jnp.pad(pairs_flat, (0, 2 * pad))
        r0 = jnp.pad(r0, (0, pad))
        k = jnp.pad(k, (0, pad))
    nchunks = e_pad // (NW * CHUNK)

    cf = coords.astype(jnp.float32)
    xb = lax.bitcast_convert_type(cf[:, 0].astype(jnp.bfloat16),
                                  jnp.uint16).astype(jnp.uint32)
    yb = lax.bitcast_convert_type(cf[:, 1].astype(jnp.bfloat16),
                                  jnp.uint16).astype(jnp.uint32)
    xy_packed = ((yb << 16) | xb).astype(jnp.int32)
    z_bits = lax.bitcast_convert_type(cf[:, 2], jnp.int32)

    mesh = plsc.VectorSubcoreMesh(core_axis_name="c", subcore_axis_name="s")
    buf = lambda: [
        pltpu.VMEM((2 * CHUNK,), jnp.int32),
        pltpu.VMEM((CHUNK,), jnp.float32),
        pltpu.VMEM((CHUNK,), jnp.float32),
        pltpu.VMEM((CHUNK,), jnp.float32),
    ]
    f = pl.kernel(
        functools.partial(_bond_kernel, nchunks),
        mesh=mesh,
        out_type=(
            jax.ShapeDtypeStruct((NW, 16), jnp.float32),
            jax.ShapeDtypeStruct((e_pad,), jnp.float32),
        ),
        scratch_types=[pltpu.VMEM((n,), jnp.int32)] + buf() + buf() + [
            pltpu.VMEM((16,), jnp.float32),
            pltpu.SemaphoreType.DMA,
            pltpu.SemaphoreType.DMA,
            pltpu.SemaphoreType.DMA,
            pltpu.SemaphoreType.DMA,
        ],
        compiler_params=pltpu.CompilerParams(
            needs_layout_passes=False, use_tc_tiling_on_sc=False),
    )
    partials, _ = f(xy_packed, z_bits, pairs_flat, r0, k)
    return jnp.sum(partials)


# speedup vs baseline: 1.0013x; 1.0013x over previous
import functools

import jax
import jax.numpy as jnp
from jax import lax
from jax.experimental import pallas as pl
from jax.experimental.pallas import tpu as pltpu
from jax.experimental.pallas import tpu_sc as plsc

NC = 2
NS = 16
NW = NC * NS
CHUNK = 2000


def _unpack_lo(w):
    return plsc.bitcast(w << 16, jnp.float32)


def _unpack_hi(w):
    return plsc.bitcast((w >> 16) << 16, jnp.float32)


def _bond_kernel(nchunks, xy_hbm, z_hbm, pairs_hbm, r0_hbm, k_hbm,
                 out_hbm, stage_hbm,
                 table_v,
                 pr0_v, dd0_v, r00_v, k0_v,
                 pr1_v, dd1_v, r01_v, k1_v,
                 acc_v, sem0, sem1, osem0, osem1):
    cid = lax.axis_index("c")
    sid = lax.axis_index("s")
    wid = sid * NC + cid
    base_e = wid * nchunks * CHUNK

    lane2 = lax.iota(jnp.int32, 16) * 2

    bufs = (
        (pr0_v, dd0_v, r00_v, k0_v, sem0, osem0),
        (pr1_v, dd1_v, r01_v, k1_v, sem1, osem1),
    )

    pltpu.sync_copy(xy_hbm, table_v)

    def issue1(ch, b):
        pr_v, _, _, _, sem, _ = bufs[b]
        eb = base_e + ch * CHUNK
        pltpu.async_copy(pairs_hbm.at[pl.ds(2 * eb, 2 * CHUNK)], pr_v, sem)

    def drain1(b):
        pr_v, _, _, _, sem, _ = bufs[b]
        pltpu.make_async_copy(pairs_hbm.at[pl.ds(0, 2 * CHUNK)], pr_v,
                              sem).wait()

    def compute1(ch, b):
        pr_v, dd_v, _, _, _, osem = bufs[b]
        eb = base_e + ch * CHUNK

        @plsc.parallel_loop(0, CHUNK // 16, unroll=5)
        def vloop(v):
            e0 = pl.multiple_of(v * 16, 16)
            p0 = e0 * 2 + lane2
            iv = plsc.load_gather(pr_v, [p0])
            jv = plsc.load_gather(pr_v, [p0 + 1])
            wi = plsc.load_gather(table_v, [iv])
            wj = plsc.load_gather(table_v, [jv])
            dx = _unpack_lo(wi) - _unpack_lo(wj)
            dy = _unpack_hi(wi) - _unpack_hi(wj)
            dd_v[pl.ds(e0, 16)] = dx * dx + dy * dy

        pltpu.async_copy(dd_v, stage_hbm.at[pl.ds(eb, CHUNK)], osem)

    def drain_out(b):
        _, dd_v, _, _, _, osem = bufs[b]
        pltpu.make_async_copy(stage_hbm.at[pl.ds(0, CHUNK)], dd_v,
                              osem).wait()

    issue1(0, 0)
    issue1(1, 1)

    def body1(p, carry):
        drain1(0)
        compute1(2 * p, 0)
        issue1(2 * p + 2, 0)
        drain1(1)
        compute1(2 * p + 1, 1)
        issue1(2 * p + 3, 1)
        drain_out(0)
        drain_out(1)
        return carry

    lax.fori_loop(0, nchunks // 2 - 1, body1, jnp.int32(0))
    drain1(0)
    compute1(nchunks - 2, 0)
    drain1(1)
    compute1(nchunks - 1, 1)
    drain_out(0)
    drain_out(1)

    pltpu.sync_copy(z_hbm, table_v)

    def issue2(ch, b):
        pr_v, dd_v, r0_v, k_v, sem, _ = bufs[b]
        eb = base_e + ch * CHUNK
        pltpu.async_copy(pairs_hbm.at[pl.ds(2 * eb, 2 * CHUNK)], pr_v, sem)
        pltpu.async_copy(stage_hbm.at[pl.ds(eb, CHUNK)], dd_v, sem)
        pltpu.async_copy(r0_hbm.at[pl.ds(eb, CHUNK)], r0_v, sem)
        pltpu.async_copy(k_hbm.at[pl.ds(eb, CHUNK)], k_v, sem)

    def drain2(b):
        pr_v, dd_v, r0_v, k_v, sem, _ = bufs[b]
        pltpu.make_async_copy(pairs_hbm.at[pl.ds(0, 2 * CHUNK)], pr_v,
                              sem).wait()
        pltpu.make_async_copy(stage_hbm.at[pl.ds(0, CHUNK)], dd_v,
                              sem).wait()
        pltpu.make_async_copy(r0_hbm.at[pl.ds(0, CHUNK)], r0_v, sem).wait()
        pltpu.make_async_copy(k_hbm.at[pl.ds(0, CHUNK)], k_v, sem).wait()

    def compute2(b, acc):
        pr_v, dd_v, r0_v, k_v, _, _ = bufs[b]

        @plsc.parallel_loop(0, CHUNK // 16, unroll=5, carry=acc)
        def vloop(v, acc):
            e0 = pl.multiple_of(v * 16, 16)
            p0 = e0 * 2 + lane2
            iv = plsc.load_gather(pr_v, [p0])
            jv = plsc.load_gather(pr_v, [p0 + 1])
            zi = plsc.bitcast(plsc.load_gather(table_v, [iv]), jnp.float32)
            zj = plsc.bitcast(plsc.load_gather(table_v, [jv]), jnp.float32)
            dz = zi - zj
            d2 = dd_v[pl.ds(e0, 16)] + dz * dz
            d2 = jnp.maximum(d2, 1e-12)
            ib = plsc.bitcast(d2, jnp.int32)
            y = plsc.bitcast(jnp.int32(0x5F3759DF) - (ib >> 1), jnp.float32)
            hx = 0.5 * d2
            y = y * (1.5 - hx * y * y)
            y = y * (1.5 - hx * y * y)
            r = d2 * y
            t = r - r0_v[pl.ds(e0, 16)]
            return acc + (t * t) * k_v[pl.ds(e0, 16)]

        return vloop

    issue2(0, 0)
    issue2(1, 1)

    def body2(p, acc):
        drain2(0)
        acc = compute2(0, acc)
        issue2(2 * p + 2, 0)
        drain2(1)
        acc = compute2(1, acc)
        issue2(2 * p + 3, 1)
        return acc

    acc = lax.fori_loop(0, nchunks // 2 - 1, body2,
                        jnp.zeros((16,), jnp.float32))
    drain2(0)
    acc = compute2(0, acc)
    drain2(1)
    acc = compute2(1, acc)

    acc_v[...] = acc * 0.5
    pltpu.sync_copy(acc_v, out_hbm.at[wid])


def kernel(coords, pairs, r0, k):
    e = pairs.shape[0]
    n = coords.shape[0]
    r0 = r0.astype(jnp.float32)
    k = k.astype(jnp.float32)
    pairs_flat = pairs.astype(jnp.int32).reshape(2 * e)

    grain = NW * CHUNK * 2
    e_pad = ((e + grain - 1) // grain) * grain
    pad = e_pad - e
    if pad:
        pairs_flat = jnp.pad(pairs_flat, (0, 2 * pad))
        r0 = jnp.pad(r0, (0, pad))
        k = jnp.pad(k, (0, pad))
    nchunks = e_pad // (NW * CHUNK)

    cf = coords.astype(jnp.float32)
    xb = lax.bitcast_convert_type(cf[:, 0].astype(jnp.bfloat16),
                                  jnp.uint16).astype(jnp.uint32)
    yb = lax.bitcast_convert_type(cf[:, 1].astype(jnp.bfloat16),
                                  jnp.uint16).astype(jnp.uint32)
    xy_packed = ((yb << 16) | xb).astype(jnp.int32)
    z_bits = lax.bitcast_convert_type(cf[:, 2], jnp.int32)

    mesh = plsc.VectorSubcoreMesh(core_axis_name="c", subcore_axis_name="s")
    buf = lambda: [
        pltpu.VMEM((2 * CHUNK,), jnp.int32),
        pltpu.VMEM((CHUNK,), jnp.float32),
        pltpu.VMEM((CHUNK,), jnp.float32),
        pltpu.VMEM((CHUNK,), jnp.float32),
    ]
    f = pl.kernel(
        functools.partial(_bond_kernel, nchunks),
        mesh=mesh,
        out_type=(
            jax.ShapeDtypeStruct((NW, 16), jnp.float32),
            jax.ShapeDtypeStruct((e_pad,), jnp.float32),
        ),
        scratch_types=[pltpu.VMEM((n,), jnp.int32)] + buf() + buf() + [
            pltpu.VMEM((16,), jnp.float32),
            pltpu.SemaphoreType.DMA,
            pltpu.SemaphoreType.DMA,
            pltpu.SemaphoreType.DMA,
            pltpu.SemaphoreType.DMA,
        ],
        compiler_params=pltpu.CompilerParams(
            needs_layout_passes=False, use_tc_tiling_on_sc=False),
    )
    partials, _ = f(xy_packed, z_bits, pairs_flat, r0, k)
    return jnp.sum(partials)


# speedup vs baseline: 28.2717x; 28.2343x over previous
import functools

import jax
import jax.numpy as jnp
from jax import lax
from jax.experimental import pallas as pl
from jax.experimental.pallas import tpu as pltpu
from jax.experimental.pallas import tpu_sc as plsc

NC = 2
NS = 16
NW = NC * NS
CHUNK = 2000


def _unpack_lo(w):
    return plsc.bitcast(w << 16, jnp.float32)


def _unpack_hi(w):
    return plsc.bitcast((w >> 16) << 16, jnp.float32)


def _bond_kernel(nchunks, xy_hbm, z_hbm, idxi_hbm, idxj_hbm, r0_hbm, k_hbm,
                 out_hbm, stage_hbm,
                 table_v,
                 pi0_v, pj0_v, dd0_v, r00_v, k0_v,
                 pi1_v, pj1_v, dd1_v, r01_v, k1_v,
                 acc_v, sem0, sem1, osem0, osem1):
    cid = lax.axis_index("c")
    sid = lax.axis_index("s")
    wid = sid * NC + cid
    base_e = wid * nchunks * CHUNK

    lane2 = lax.iota(jnp.int32, 16) * 2

    bufs = (
        (pi0_v, pj0_v, dd0_v, r00_v, k0_v, sem0, osem0),
        (pi1_v, pj1_v, dd1_v, r01_v, k1_v, sem1, osem1),
    )

    pltpu.sync_copy(xy_hbm, table_v)

    def issue1(ch, b):
        pi_v, pj_v, _, _, _, sem, _ = bufs[b]
        eb = base_e + ch * CHUNK
        pltpu.async_copy(idxi_hbm.at[pl.ds(eb, CHUNK)], pi_v, sem)
        pltpu.async_copy(idxj_hbm.at[pl.ds(eb, CHUNK)], pj_v, sem)

    def drain1(b):
        pi_v, pj_v, _, _, _, sem, _ = bufs[b]
        pltpu.make_async_copy(idxi_hbm.at[pl.ds(0, CHUNK)], pi_v,
                              sem).wait()
        pltpu.make_async_copy(idxj_hbm.at[pl.ds(0, CHUNK)], pj_v,
                              sem).wait()

    def compute1(ch, b):
        pi_v, pj_v, dd_v, _, _, _, osem = bufs[b]
        eb = base_e + ch * CHUNK

        @plsc.parallel_loop(0, CHUNK // 16, unroll=5)
        def vloop(v):
            e0 = pl.multiple_of(v * 16, 16)
            iv = pi_v[pl.ds(e0, 16)]
            jv = pj_v[pl.ds(e0, 16)]
            wi = plsc.load_gather(table_v, [iv])
            wj = plsc.load_gather(table_v, [jv])
            dx = _unpack_lo(wi) - _unpack_lo(wj)
            dy = _unpack_hi(wi) - _unpack_hi(wj)
            dd_v[pl.ds(e0, 16)] = dx * dx + dy * dy

        pltpu.async_copy(dd_v, stage_hbm.at[pl.ds(eb, CHUNK)], osem)

    def drain_out(b):
        _, _, dd_v, _, _, _, osem = bufs[b]
        pltpu.make_async_copy(stage_hbm.at[pl.ds(0, CHUNK)], dd_v,
                              osem).wait()

    issue1(0, 0)
    issue1(1, 1)

    def body1(p, carry):
        drain1(0)
        compute1(2 * p, 0)
        issue1(2 * p + 2, 0)
        drain1(1)
        compute1(2 * p + 1, 1)
        issue1(2 * p + 3, 1)
        drain_out(0)
        drain_out(1)
        return carry

    lax.fori_loop(0, nchunks // 2 - 1, body1, jnp.int32(0))
    drain1(0)
    compute1(nchunks - 2, 0)
    drain1(1)
    compute1(nchunks - 1, 1)
    drain_out(0)
    drain_out(1)

    pltpu.sync_copy(z_hbm, table_v)

    def issue2(ch, b):
        pi_v, pj_v, dd_v, r0_v, k_v, sem, _ = bufs[b]
        eb = base_e + ch * CHUNK
        pltpu.async_copy(idxi_hbm.at[pl.ds(eb, CHUNK)], pi_v, sem)
        pltpu.async_copy(idxj_hbm.at[pl.ds(eb, CHUNK)], pj_v, sem)
        pltpu.async_copy(stage_hbm.at[pl.ds(eb, CHUNK)], dd_v, sem)
        pltpu.async_copy(r0_hbm.at[pl.ds(eb, CHUNK)], r0_v, sem)
        pltpu.async_copy(k_hbm.at[pl.ds(eb, CHUNK)], k_v, sem)

    def drain2(b):
        pi_v, pj_v, dd_v, r0_v, k_v, sem, _ = bufs[b]
        pltpu.make_async_copy(idxi_hbm.at[pl.ds(0, CHUNK)], pi_v,
                              sem).wait()
        pltpu.make_async_copy(idxj_hbm.at[pl.ds(0, CHUNK)], pj_v,
                              sem).wait()
        pltpu.make_async_copy(stage_hbm.at[pl.ds(0, CHUNK)], dd_v,
                              sem).wait()
        pltpu.make_async_copy(r0_hbm.at[pl.ds(0, CHUNK)], r0_v, sem).wait()
        pltpu.make_async_copy(k_hbm.at[pl.ds(0, CHUNK)], k_v, sem).wait()

    def compute2(b, acc):
        pi_v, pj_v, dd_v, r0_v, k_v, _, _ = bufs[b]

        @plsc.parallel_loop(0, CHUNK // 16, unroll=5, carry=acc)
        def vloop(v, acc):
            e0 = pl.multiple_of(v * 16, 16)
            iv = pi_v[pl.ds(e0, 16)]
            jv = pj_v[pl.ds(e0, 16)]
            zi = plsc.bitcast(plsc.load_gather(table_v, [iv]), jnp.float32)
            zj = plsc.bitcast(plsc.load_gather(table_v, [jv]), jnp.float32)
            dz = zi - zj
            d2 = dd_v[pl.ds(e0, 16)] + dz * dz
            d2 = jnp.maximum(d2, 1e-12)
            ib = plsc.bitcast(d2, jnp.int32)
            y = plsc.bitcast(jnp.int32(0x5F3759DF) - (ib >> 1), jnp.float32)
            hx = 0.5 * d2
            y = y * (1.5 - hx * y * y)
            y = y * (1.5 - hx * y * y)
            r = d2 * y
            t = r - r0_v[pl.ds(e0, 16)]
            return acc + (t * t) * k_v[pl.ds(e0, 16)]

        return vloop

    issue2(0, 0)
    issue2(1, 1)

    def body2(p, acc):
        drain2(0)
        acc = compute2(0, acc)
        issue2(2 * p + 2, 0)
        drain2(1)
        acc = compute2(1, acc)
        issue2(2 * p + 3, 1)
        return acc

    acc = lax.fori_loop(0, nchunks // 2 - 1, body2,
                        jnp.zeros((16,), jnp.float32))
    drain2(0)
    acc = compute2(0, acc)
    drain2(1)
    acc = compute2(1, acc)

    acc_v[...] = acc * 0.5
    pltpu.sync_copy(acc_v, out_hbm.at[wid])


def kernel(coords, pairs, r0, k):
    e = pairs.shape[0]
    n = coords.shape[0]
    r0 = r0.astype(jnp.float32)
    k = k.astype(jnp.float32)
    idx_i = pairs[:, 0].astype(jnp.int32)
    idx_j = pairs[:, 1].astype(jnp.int32)

    grain = NW * CHUNK * 2
    e_pad = ((e + grain - 1) // grain) * grain
    pad = e_pad - e
    if pad:
        idx_i = jnp.pad(idx_i, (0, pad))
        idx_j = jnp.pad(idx_j, (0, pad))
        r0 = jnp.pad(r0, (0, pad))
        k = jnp.pad(k, (0, pad))
    nchunks = e_pad // (NW * CHUNK)

    cf = coords.astype(jnp.float32)
    xb = lax.bitcast_convert_type(cf[:, 0].astype(jnp.bfloat16),
                                  jnp.uint16).astype(jnp.uint32)
    yb = lax.bitcast_convert_type(cf[:, 1].astype(jnp.bfloat16),
                                  jnp.uint16).astype(jnp.uint32)
    xy_packed = ((yb << 16) | xb).astype(jnp.int32)
    z_bits = lax.bitcast_convert_type(cf[:, 2], jnp.int32)

    mesh = plsc.VectorSubcoreMesh(core_axis_name="c", subcore_axis_name="s")
    buf = lambda: [
        pltpu.VMEM((CHUNK,), jnp.int32),
        pltpu.VMEM((CHUNK,), jnp.int32),
        pltpu.VMEM((CHUNK,), jnp.float32),
        pltpu.VMEM((CHUNK,), jnp.float32),
        pltpu.VMEM((CHUNK,), jnp.float32),
    ]
    f = pl.kernel(
        functools.partial(_bond_kernel, nchunks),
        mesh=mesh,
        out_type=(
            jax.ShapeDtypeStruct((NW, 16), jnp.float32),
            jax.ShapeDtypeStruct((e_pad,), jnp.float32),
        ),
        scratch_types=[pltpu.VMEM((n,), jnp.int32)] + buf() + buf() + [
            pltpu.VMEM((16,), jnp.float32),
            pltpu.SemaphoreType.DMA,
            pltpu.SemaphoreType.DMA,
            pltpu.SemaphoreType.DMA,
            pltpu.SemaphoreType.DMA,
        ],
        compiler_params=pltpu.CompilerParams(
            needs_layout_passes=False, use_tc_tiling_on_sc=False),
    )
    partials, _ = f(xy_packed, z_bits, idx_i, idx_j, r0, k)
    return jnp.sum(partials)
